# Initial kernel scaffold; baseline (speedup 1.0000x reference)
import jax
import jax.numpy as jnp
from jax.experimental import pallas as pl


def kernel(inputs, targets):
    raise NotImplementedError("write your pallas kernel here")


# speedup vs baseline: 20.2451x; 20.2451x over previous
import jax
import jax.numpy as jnp
from jax import lax
from jax.experimental import pallas as pl
from jax.experimental.pallas import tpu as pltpu

_BIG = 1e8


def _dtw_kernel(x_ref, y_ref, out_ref, s0_ref, s1_ref, r1_ref, r2_ref):
    BB, N, D = x_ref.shape
    f32 = jnp.float32
    big = f32(_BIG)

    for b in range(BB):
        xb = x_ref[b]
        yb = y_ref[b]
        x2 = jnp.sum(xb * xb, axis=1, keepdims=True)
        y2 = jnp.sum(yb * yb, axis=1, keepdims=True)
        ones = jnp.ones((N, 1), dtype=f32)
        xh = jnp.concatenate([xb, x2, ones], axis=1)
        yh = jnp.concatenate([-2.0 * yb, ones, y2], axis=1)
        nt = N // 4
        for t in range(4):
            s0_ref[b, t * nt:(t + 1) * nt, :] = lax.dot_general(
                yh[t * nt:(t + 1) * nt, :], xh,
                (((1,), (1,)), ((), ())),
                preferred_element_type=f32)

    nbits = N.bit_length() - 1
    lane = lax.broadcasted_iota(jnp.int32, (1, 1, N), 2)
    bufs = [s0_ref, s1_ref]
    for k in range(nbits):
        src = bufs[k % 2]
        dst = bufs[(k + 1) % 2]
        s = 1 << k
        cur = src[...]
        rolled = jnp.concatenate([cur[:, N - s:, :], cur[:, :N - s, :]], axis=1)
        mask = ((lane >> k) & 1) == 1
        dst[...] = jnp.where(mask, rolled, cur)
    sfin = bufs[nbits % 2]

    r1_ref[...] = jnp.full((BB, N), big, dtype=f32)
    r2_ref[...] = jnp.full((BB, N), big, dtype=f32)
    av = lax.broadcasted_iota(jnp.int32, (BB, N), 1)
    pad_big = jnp.full((BB, 1), big, dtype=f32)

    def body(q, carry):
        c0 = pl.multiple_of((q * 8) & (N - 1), 8)
        slab = sfin[:, pl.ds(c0, 8), :]
        r1 = r1_ref[...]
        r2 = r2_ref[...]
        for kk in range(8):
            d = q * 8 + (kk + 2)
            cost = slab[:, kk, :]
            fill2 = jnp.where(d == 2, f32(0.0), big)
            r1u = jnp.concatenate([pad_big, r1[:, :-1]], axis=1)
            r2u = jnp.concatenate(
                [jnp.broadcast_to(fill2, (BB, 1)), r2[:, :-1]], axis=1)
            m = jnp.minimum(jnp.minimum(r1u, r1), r2u)
            ssum = (jnp.exp(m - r1u) + jnp.exp(m - r1) + jnp.exp(m - r2u))
            smin = m - jnp.log(ssum)
            valid = (av <= d - 2) & (av >= d - (N + 1))
            new = jnp.where(valid, cost + smin, big)
            r2 = r1
            r1 = new
        r1_ref[...] = r1
        r2_ref[...] = r2
        return carry

    lax.fori_loop(0, (2 * N) // 8, body, 0)
    out_ref[...] = r2_ref[:, N - 1:N]


def _dtw_batch(x, y, bb, interpret=False):
    B, N, D = x.shape
    grid = (B // bb,)
    return pl.pallas_call(
        _dtw_kernel,
        out_shape=jax.ShapeDtypeStruct((B, 1), jnp.float32),
        grid=grid,
        in_specs=[
            pl.BlockSpec((bb, N, D), lambda p: (p, 0, 0)),
            pl.BlockSpec((bb, N, D), lambda p: (p, 0, 0)),
        ],
        out_specs=pl.BlockSpec((bb, 1), lambda p: (p, 0)),
        scratch_shapes=[
            pltpu.VMEM((bb, N, N), jnp.float32),
            pltpu.VMEM((bb, N, N), jnp.float32),
            pltpu.VMEM((bb, N), jnp.float32),
            pltpu.VMEM((bb, N), jnp.float32),
        ],
        compiler_params=pltpu.CompilerParams(
            dimension_semantics=("parallel",),
            vmem_limit_bytes=50 * 1024 * 1024,
        ),
        name="soft_dtw",
        interpret=interpret,
    )(x, y)


def kernel(inputs, targets):
    r = _dtw_batch(inputs, targets, bb=8)
    return jnp.mean(r)


# speedup vs baseline: 35.3696x; 1.7471x over previous
import jax
import jax.numpy as jnp
from jax import lax
from jax.experimental import pallas as pl
from jax.experimental.pallas import tpu as pltpu

_BIG = 1e8


def _dtw_kernel(x_ref, y_ref, out_ref, s0_ref, s1_ref, r1_ref, r2_ref):
    BB, N, D = x_ref.shape
    f32 = jnp.float32
    big = f32(_BIG)

    for b in range(BB):
        xb = x_ref[b]
        yb = y_ref[b]
        x2 = jnp.sum(xb * xb, axis=1, keepdims=True)
        y2 = jnp.sum(yb * yb, axis=1, keepdims=True)
        ones = jnp.ones((N, 1), dtype=f32)
        xh = jnp.concatenate([xb, x2, ones], axis=1)
        yh = jnp.concatenate([-2.0 * yb, ones, y2], axis=1)
        nt = N // 4
        for t in range(4):
            s0_ref[b, t * nt:(t + 1) * nt, :] = lax.dot_general(
                yh[t * nt:(t + 1) * nt, :], xh,
                (((1,), (1,)), ((), ())),
                preferred_element_type=f32)

    nbits = N.bit_length() - 1
    lane = lax.broadcasted_iota(jnp.int32, (1, 1, N), 2)
    bufs = [s0_ref, s1_ref]
    for k in range(nbits):
        src = bufs[k % 2]
        dst = bufs[(k + 1) % 2]
        s = 1 << k
        cur = src[...]
        rolled = jnp.concatenate([cur[:, N - s:, :], cur[:, :N - s, :]], axis=1)
        mask = ((lane >> k) & 1) == 1
        dst[...] = jnp.where(mask, rolled, cur)
    sfin = bufs[nbits % 2]

    av = lax.broadcasted_iota(jnp.int32, (BB, N), 1)
    pad_big = jnp.full((BB, 1), big, dtype=f32)
    r1_ref[...] = jnp.full((BB, N), big, dtype=f32)
    r2_ref[...] = jnp.where(av == 0, f32(0.0), big)

    def make_body(phase):
        def step(e, cost, r1, r2u):
            r1u = jnp.concatenate([pad_big, r1[:, :-1]], axis=1)
            m = jnp.minimum(jnp.minimum(r1u, r1), r2u)
            ssum = jnp.exp(m - r1u) + jnp.exp(m - r1) + jnp.exp(m - r2u)
            smin = m - jnp.log(ssum)
            if phase == 0:
                valid = av <= e
            else:
                valid = av >= e - (N - 1)
            new = jnp.where(valid, cost + smin, big)
            return new, r1u

        def body(q, carry):
            c0 = pl.multiple_of((q * 8) & (N - 1), 8)
            slab = sfin[:, pl.ds(c0, 8), :]
            r1 = r1_ref[...]
            r2u = r2_ref[...]
            for kk in range(8):
                r1, r2u = step(q * 8 + kk, slab[:, kk, :], r1, r2u)
            r1_ref[...] = r1
            r2_ref[...] = r2u
            return carry

        return body

    nq = (2 * N) // 8
    lax.fori_loop(0, nq // 2, make_body(0), 0)
    lax.fori_loop(nq // 2, nq - 1, make_body(1), 0)
    slab = sfin[:, N - 8:N, :]
    r1 = r1_ref[...]
    r2u = r2_ref[...]
    def stepf(e, cost, r1, r2u):
        r1u = jnp.concatenate([pad_big, r1[:, :-1]], axis=1)
        m = jnp.minimum(jnp.minimum(r1u, r1), r2u)
        ssum = jnp.exp(m - r1u) + jnp.exp(m - r1) + jnp.exp(m - r2u)
        smin = m - jnp.log(ssum)
        valid = av >= e - (N - 1)
        return jnp.where(valid, cost + smin, big), r1u
    for kk in range(7):
        r1, r2u = stepf(2 * N - 8 + kk, slab[:, kk, :], r1, r2u)
    out_ref[...] = r1[:, N - 1:N]


def _dtw_batch(x, y, bb, interpret=False):
    B, N, D = x.shape
    grid = (B // bb,)
    return pl.pallas_call(
        _dtw_kernel,
        out_shape=jax.ShapeDtypeStruct((B, 1), jnp.float32),
        grid=grid,
        in_specs=[
            pl.BlockSpec((bb, N, D), lambda p: (p, 0, 0)),
            pl.BlockSpec((bb, N, D), lambda p: (p, 0, 0)),
        ],
        out_specs=pl.BlockSpec((bb, 1), lambda p: (p, 0)),
        scratch_shapes=[
            pltpu.VMEM((bb, N, N), jnp.float32),
            pltpu.VMEM((bb, N, N), jnp.float32),
            pltpu.VMEM((bb, N), jnp.float32),
            pltpu.VMEM((bb, N), jnp.float32),
        ],
        compiler_params=pltpu.CompilerParams(
            dimension_semantics=("parallel",),
            vmem_limit_bytes=50 * 1024 * 1024,
        ),
        name="soft_dtw",
        interpret=interpret,
    )(x, y)


def kernel(inputs, targets):
    r = _dtw_batch(inputs, targets, bb=16)
    return jnp.mean(r)
